# Initial kernel scaffold; baseline (speedup 1.0000x reference)
import jax
import jax.numpy as jnp
from jax.experimental import pallas as pl


def kernel(logits, labels):
    raise NotImplementedError("write your pallas kernel here")


# speedup vs baseline: 2.1820x; 2.1820x over previous
import numpy as np
import jax
import jax.numpy as jnp
from jax.experimental import pallas as pl
from jax.experimental.pallas import tpu as pltpu

_N_BINS = 15


def _ece_kernel(x_ref, lab_ref, out_ref, acc_ref, *, nblocks, n, c):
    i = pl.program_id(0)

    @pl.when(i == 0)
    def _init():
        acc_ref[...] = jnp.zeros_like(acc_ref)

    x = x_ref[...]
    lab = lab_ref[...]
    rowmax = jnp.max(x, axis=1, keepdims=True)
    e = jnp.exp(x - rowmax)
    rowsum = jnp.sum(e, axis=1, keepdims=True)
    s = e * (1.0 / rowsum)

    classes = jax.lax.broadcasted_iota(jnp.int32, (1, c), 1)
    onehot = lab == classes
    q = jnp.where(onehot, s - 1.0, s)
    q = jnp.where(s > 0.0, q, 0.0)

    bounds = np.linspace(0.0, 1.0, _N_BINS + 1).astype(np.float32)
    for k in range(1, _N_BINS):
        t = jnp.sum(jnp.where(s <= bounds[k], q, 0.0), axis=0, keepdims=True)
        acc_ref[k - 1 : k, :] += t
    acc_ref[_N_BINS - 1 : _N_BINS, :] += jnp.sum(q, axis=0, keepdims=True)

    @pl.when(i == nblocks - 1)
    def _fin():
        T = acc_ref[...]
        total = jnp.sum(jnp.abs(T[0:1, :])) + jnp.sum(
            jnp.abs(T[1:, :] - T[:-1, :])
        )
        out_ref[...] = (total / (n * c)).reshape(1, 1)


def kernel(logits, labels):
    n, c = logits.shape
    block_rows = 4000
    nblocks = n // block_rows
    labels2d = labels.reshape(n, 1)

    out = pl.pallas_call(
        lambda x, l, o, a: _ece_kernel(x, l, o, a, nblocks=nblocks, n=n, c=c),
        grid=(nblocks,),
        in_specs=[
            pl.BlockSpec((block_rows, c), lambda i: (i, 0)),
            pl.BlockSpec((block_rows, 1), lambda i: (i, 0)),
        ],
        out_specs=pl.BlockSpec((1, 1), lambda i: (0, 0)),
        out_shape=jax.ShapeDtypeStruct((1, 1), jnp.float32),
        scratch_shapes=[pltpu.VMEM((_N_BINS, c), jnp.float32)],
        compiler_params=pltpu.CompilerParams(
            dimension_semantics=("arbitrary",)
        ),
    )(logits, labels2d)
    return out.reshape(())
